# baseline (speedup 1.0000x reference)
import functools

import jax
import jax.numpy as jnp
from jax import lax
from jax.experimental import pallas as pl
from jax.experimental.pallas import tpu as pltpu
from jax.experimental.pallas import tpu_sc as plsc

B = 16384
D = 32
U = 1000000
SR = U // 4
L = 16
NC = 2
NS = 16
NW = NC * NS
BPW = B // NW
CHUNK = 128
NCH = BPW // CHUNK


def _gather_extract(tab_hbm, sidx, selv, stage, rows, sem):
    def fire(c, carry):
        pltpu.async_copy(
            tab_hbm.at[sidx.at[c]],
            stage.at[pl.ds(c * CHUNK, CHUNK)], sem)
        return carry

    lax.fori_loop(0, NCH, fire, 0)

    def drain(c, carry):
        pltpu.make_async_copy(
            tab_hbm.at[pl.ds(0, CHUNK)],
            stage.at[pl.ds(c * CHUNK, CHUNK)], sem).wait()
        return carry

    lax.fori_loop(0, NCH, drain, 0)

    def extract(g, carry):
        sel = selv[g // (CHUNK // L), pl.ds((g % (CHUNK // L)) * L, L)]
        for j in range(L):
            slot = g * L + j
            off = sel[j] * D
            rows[slot, pl.ds(0, L)] = stage[slot, pl.ds(off, L)]
            rows[slot, pl.ds(L, L)] = stage[slot, pl.ds(off + L, L)]
        return carry

    lax.fori_loop(0, BPW // L, extract, 0)


def _mf_body(sidx_u_hbm, sel_u_hbm, sidx_i_hbm, sel_i_hbm,
             utab_hbm, itab_hbm, wb_hbm,
             out_hbm,
             sidx_u, sel_u, sidx_i, sel_i, stage, urows, irows,
             outb, wbv, sem):
    wid = lax.axis_index("s") * NC + lax.axis_index("c")
    base = wid * BPW

    pltpu.sync_copy(sidx_u_hbm.at[wid], sidx_u)
    pltpu.sync_copy(sel_u_hbm.at[wid], sel_u)
    pltpu.sync_copy(sidx_i_hbm.at[wid], sidx_i)
    pltpu.sync_copy(sel_i_hbm.at[wid], sel_i)
    pltpu.sync_copy(wb_hbm, wbv)

    _gather_extract(utab_hbm, sidx_u, sel_u, stage, urows, sem)
    _gather_extract(itab_hbm, sidx_i, sel_i, stage, irows, sem)

    wv = wbv[0, :]
    bv = wbv[1, :]
    lane = lax.iota(jnp.int32, L)

    def body(g, carry):
        acc = jnp.zeros((L,), jnp.float32)
        for j in range(L):
            r = g * L + j
            u0 = urows[r, pl.ds(0, L)]
            u1 = urows[r, pl.ds(L, L)]
            i0 = irows[r, pl.ds(0, L)]
            i1 = irows[r, pl.ds(L, L)]
            p = u0 * i0 + u1 * i1
            s = jnp.sum(p)
            acc = jnp.where(lane == j, jnp.broadcast_to(s, (L,)), acc)
        outb[pl.ds(g * L, L)] = acc * wv + bv
        return carry

    lax.fori_loop(0, BPW // L, body, 0)

    pltpu.sync_copy(outb, out_hbm.at[pl.ds(base, BPW)])


_mf = functools.partial(
    pl.kernel,
    out_type=jax.ShapeDtypeStruct((B,), jnp.float32),
    mesh=plsc.VectorSubcoreMesh(core_axis_name="c", subcore_axis_name="s"),
    compiler_params=pltpu.CompilerParams(
        needs_layout_passes=False, use_tc_tiling_on_sc=False),
    scratch_types=[
        pltpu.VMEM((NCH, CHUNK), jnp.int32),
        pltpu.VMEM((NCH, CHUNK), jnp.int32),
        pltpu.VMEM((NCH, CHUNK), jnp.int32),
        pltpu.VMEM((NCH, CHUNK), jnp.int32),
        pltpu.VMEM((BPW, CHUNK), jnp.float32),
        pltpu.VMEM((BPW, D), jnp.float32),
        pltpu.VMEM((BPW, D), jnp.float32),
        pltpu.VMEM((BPW,), jnp.float32),
        pltpu.VMEM((2, L), jnp.float32),
        pltpu.SemaphoreType.DMA,
    ],
)(_mf_body)


def kernel(user_ids, item_ids, user_table, item_table, dense_w, dense_b):
    uid = user_ids.astype(jnp.int32)
    iid = item_ids.astype(jnp.int32)
    sidx_u = (uid // 4).reshape(NW, NCH, CHUNK)
    sel_u = (uid % 4).reshape(NW, NCH, CHUNK)
    sidx_i = (iid // 4).reshape(NW, NCH, CHUNK)
    sel_i = (iid % 4).reshape(NW, NCH, CHUNK)
    w = jnp.broadcast_to(dense_w.reshape(()), (L,)).astype(jnp.float32)
    b = jnp.broadcast_to(dense_b.reshape(()), (L,)).astype(jnp.float32)
    wb = jnp.stack([w, b])
    utab = user_table.reshape(SR, CHUNK)
    itab = item_table.reshape(SR, CHUNK)
    out = _mf(sidx_u, sel_u, sidx_i, sel_i, utab, itab, wb)
    return out.reshape(B, 1)


# speedup vs baseline: 1.0208x; 1.0208x over previous
import functools

import jax
import jax.numpy as jnp
from jax import lax
from jax.experimental import pallas as pl
from jax.experimental.pallas import tpu as pltpu
from jax.experimental.pallas import tpu_sc as plsc

B = 16384
D = 32
L = 16
NC = 2
NS = 16
NW = NC * NS
BPW = B // NW
CHUNK = 128
NCH = BPW // CHUNK


def _mf_body(uid_hbm, iid_hbm, utab_hbm, itab_hbm, wb_hbm,
             out_hbm,
             idx_u, idx_i, urows, irows, outb, wbv, sem):
    wid = lax.axis_index("s") * NC + lax.axis_index("c")
    base = wid * BPW

    pltpu.sync_copy(uid_hbm.at[wid], idx_u)
    pltpu.sync_copy(iid_hbm.at[wid], idx_i)
    pltpu.sync_copy(wb_hbm, wbv)

    copies = []
    for c in range(NCH):
        copies.append(pltpu.async_copy(
            utab_hbm.at[idx_u.at[c]], urows.at[pl.ds(c * CHUNK, CHUNK)], sem))
        copies.append(pltpu.async_copy(
            itab_hbm.at[idx_i.at[c]], irows.at[pl.ds(c * CHUNK, CHUNK)], sem))
    for cp in copies:
        cp.wait()

    wv = wbv[0, :]
    bv = wbv[1, :]
    lane = lax.iota(jnp.int32, L)

    def body(g, carry):
        acc = jnp.zeros((L,), jnp.float32)
        for j in range(L):
            r = g * L + j
            u0 = urows[r, pl.ds(0, L)]
            u1 = urows[r, pl.ds(L, L)]
            i0 = irows[r, pl.ds(0, L)]
            i1 = irows[r, pl.ds(L, L)]
            p = u0 * i0 + u1 * i1
            s = jnp.sum(p)
            acc = jnp.where(lane == j, jnp.broadcast_to(s, (L,)), acc)
        outb[pl.ds(g * L, L)] = acc * wv + bv
        return carry

    lax.fori_loop(0, BPW // L, body, 0)

    pltpu.sync_copy(outb, out_hbm.at[pl.ds(base, BPW)])


_mf = functools.partial(
    pl.kernel,
    out_type=jax.ShapeDtypeStruct((B,), jnp.float32),
    mesh=plsc.VectorSubcoreMesh(core_axis_name="c", subcore_axis_name="s"),
    compiler_params=pltpu.CompilerParams(
        needs_layout_passes=False, use_tc_tiling_on_sc=False),
    scratch_types=[
        pltpu.VMEM((NCH, CHUNK), jnp.int32),
        pltpu.VMEM((NCH, CHUNK), jnp.int32),
        pltpu.VMEM((BPW, D), jnp.float32),
        pltpu.VMEM((BPW, D), jnp.float32),
        pltpu.VMEM((BPW,), jnp.float32),
        pltpu.VMEM((2, L), jnp.float32),
        pltpu.SemaphoreType.DMA,
    ],
)(_mf_body)


def kernel(user_ids, item_ids, user_table, item_table, dense_w, dense_b):
    uid = user_ids.astype(jnp.int32).reshape(NW, NCH, CHUNK)
    iid = item_ids.astype(jnp.int32).reshape(NW, NCH, CHUNK)
    w = jnp.broadcast_to(dense_w.reshape(()), (L,)).astype(jnp.float32)
    b = jnp.broadcast_to(dense_b.reshape(()), (L,)).astype(jnp.float32)
    wb = jnp.stack([w, b])
    out = _mf(uid, iid, user_table, item_table, wb)
    return out.reshape(B, 1)
